# speedup vs baseline: 3.0891x; 3.0891x over previous
import functools

import jax
import jax.numpy as jnp
from jax import lax
from jax.experimental import pallas as pl
from jax.experimental.pallas import tpu as pltpu
from jax.experimental.pallas import tpu_sc as plsc

_MASK_TOKEN_ID = 32
_MASK_RATIO_TRAIN = 0.15 * 0.8

_NC = 2
_NS = 16
_NW = _NC * _NS

_CHUNK = 32
_NBUF = 3


def _prep_body(cu_ref, ids_ref, tab_ref, tabout_ref, comb_ref):
    B = tabout_ref.shape[0]
    V = tab_ref.shape[0]
    ids = ids_ref[...]
    pos = lax.broadcasted_iota(jnp.int32, ids.shape, 1)
    seg = jnp.zeros(ids.shape, jnp.int32)
    for j in range(1, B):
        seg = seg + jnp.where(pos >= cu_ref[j], 1, 0)
    comb_ref[...] = seg * V + ids
    masked = jnp.where(ids == _MASK_TOKEN_ID, 1.0, 0.0)
    tab = tab_ref[...]
    row = lax.broadcasted_iota(jnp.int32, tab.shape, 0)
    tabz = jnp.where(row == _MASK_TOKEN_ID, 0.0, tab)
    for b in range(B):
        nm = jnp.sum(jnp.where(seg == b, masked, 0.0))
        ln = (cu_ref[b + 1] - cu_ref[b]).astype(jnp.float32)
        scale = (1.0 - _MASK_RATIO_TRAIN) / (1.0 - nm / ln)
        tabout_ref[b] = tabz * scale


def _gather_body(nchunk, tok_per_w, comb_hbm, tab_hbm, out_hbm, idx_v, *rest):
    bufs = rest[:_NBUF]
    gsems = rest[_NBUF:2 * _NBUF]
    ssems = rest[2 * _NBUF:3 * _NBUF]
    wid = lax.axis_index("c") * _NS + lax.axis_index("s")
    base = wid * tok_per_w
    pltpu.sync_copy(comb_hbm.at[wid], idx_v)
    gcp = [None] * nchunk
    scp = [None] * nchunk
    for g in range(_NBUF):
        gcp[g] = pltpu.async_copy(tab_hbm.at[idx_v.at[g]], bufs[g], gsems[g])
    for g in range(nchunk):
        r = g % _NBUF
        gcp[g].wait()
        scp[g] = pltpu.async_copy(
            bufs[r], out_hbm.at[pl.ds(base + g * _CHUNK, _CHUNK)], ssems[r])
        h = g + _NBUF
        if h < nchunk:
            scp[g].wait()
            gcp[h] = pltpu.async_copy(tab_hbm.at[idx_v.at[h]], bufs[r], gsems[r])
    for g in range(nchunk - _NBUF, nchunk):
        scp[g].wait()


def kernel(input_ids, cu_seq_lens_q, cu_seq_lens_k, max_length_q, max_length_k, word_embeddings):
    T = input_ids.shape[1]
    V, D = word_embeddings.shape
    B = cu_seq_lens_q.shape[0] - 1
    tok_per_w = T // _NW
    nchunk = tok_per_w // _CHUNK
    assert tok_per_w * _NW == T and nchunk * _CHUNK == tok_per_w

    tabout, comb = pl.pallas_call(
        _prep_body,
        out_shape=(
            jax.ShapeDtypeStruct((B, V, D), jnp.float32),
            jax.ShapeDtypeStruct((1, T), jnp.int32),
        ),
        in_specs=[
            pl.BlockSpec(memory_space=pltpu.SMEM),
            pl.BlockSpec(memory_space=pltpu.VMEM),
            pl.BlockSpec(memory_space=pltpu.VMEM),
        ],
        out_specs=(
            pl.BlockSpec(memory_space=pltpu.VMEM),
            pl.BlockSpec(memory_space=pltpu.VMEM),
        ),
    )(cu_seq_lens_q, input_ids, word_embeddings)

    scaled = tabout.reshape(B * V, D)
    comb3 = comb.reshape(_NW, nchunk, _CHUNK)

    gather = pl.kernel(
        functools.partial(_gather_body, nchunk, tok_per_w),
        out_type=jax.ShapeDtypeStruct((T, D), jnp.float32),
        mesh=plsc.VectorSubcoreMesh(core_axis_name="c", subcore_axis_name="s"),
        scratch_types=(
            [pltpu.VMEM((nchunk, _CHUNK), jnp.int32)]
            + [pltpu.VMEM((_CHUNK, D), jnp.float32) for _ in range(_NBUF)]
            + [pltpu.SemaphoreType.DMA for _ in range(2 * _NBUF)]
        ),
    )
    out = gather(comb3, scaled)
    return out.reshape(1, T, D)


# speedup vs baseline: 3.1001x; 1.0036x over previous
import functools

import jax
import jax.numpy as jnp
from jax import lax
from jax.experimental import pallas as pl
from jax.experimental.pallas import tpu as pltpu
from jax.experimental.pallas import tpu_sc as plsc

_MASK_TOKEN_ID = 32
_MASK_RATIO_TRAIN = 0.15 * 0.8

_NC = 2
_NS = 16
_NW = _NC * _NS

_CHUNK = 32
_NBUF = 3


def _prep_body(cu_ref, ids_ref, tab_ref, tabout_ref, comb_ref):
    V = tab_ref.shape[0]
    B = tabout_ref.shape[0] // V
    ids = ids_ref[...]
    pos = lax.broadcasted_iota(jnp.int32, ids.shape, 1)
    seg = jnp.zeros(ids.shape, jnp.int32)
    for j in range(1, B):
        seg = seg + jnp.where(pos >= cu_ref[j], 1, 0)
    comb_ref[...] = seg * V + ids
    masked = jnp.where(ids == _MASK_TOKEN_ID, 1.0, 0.0)
    tab = tab_ref[...]
    row = lax.broadcasted_iota(jnp.int32, tab.shape, 0)
    tabz = jnp.where(row == _MASK_TOKEN_ID, 0.0, tab)
    for b in range(B):
        nm = jnp.sum(jnp.where(seg == b, masked, 0.0))
        ln = (cu_ref[b + 1] - cu_ref[b]).astype(jnp.float32)
        scale = (1.0 - _MASK_RATIO_TRAIN) / (1.0 - nm / ln)
        tabout_ref[pl.ds(b * V, V), :] = tabz * scale


def _gather_body(nchunk, tok_per_w, comb_hbm, tab_hbm, out_hbm, idx_v, *rest):
    bufs = rest[:_NBUF]
    gsems = rest[_NBUF:2 * _NBUF]
    ssems = rest[2 * _NBUF:3 * _NBUF]
    wid = lax.axis_index("c") * _NS + lax.axis_index("s")
    base = wid * tok_per_w
    pltpu.sync_copy(comb_hbm.at[0, pl.ds(base, tok_per_w)], idx_v)

    def idxr(g):
        return idx_v.at[pl.ds(g * _CHUNK, _CHUNK)]

    gcp = [None] * nchunk
    scp = [None] * nchunk
    for g in range(_NBUF):
        gcp[g] = pltpu.async_copy(tab_hbm.at[idxr(g)], bufs[g], gsems[g])
    for g in range(nchunk):
        r = g % _NBUF
        gcp[g].wait()
        scp[g] = pltpu.async_copy(
            bufs[r], out_hbm.at[pl.ds(base + g * _CHUNK, _CHUNK)], ssems[r])
        h = g - 1 + _NBUF
        if g >= 1 and h < nchunk:
            rr = (g - 1) % _NBUF
            scp[g - 1].wait()
            gcp[h] = pltpu.async_copy(tab_hbm.at[idxr(h)], bufs[rr], gsems[rr])
    for g in range(max(0, nchunk - _NBUF), nchunk):
        scp[g].wait()


def kernel(input_ids, cu_seq_lens_q, cu_seq_lens_k, max_length_q, max_length_k, word_embeddings):
    T = input_ids.shape[1]
    V, D = word_embeddings.shape
    B = cu_seq_lens_q.shape[0] - 1
    tok_per_w = T // _NW
    nchunk = tok_per_w // _CHUNK
    assert tok_per_w * _NW == T and nchunk * _CHUNK == tok_per_w

    scaled, comb = pl.pallas_call(
        _prep_body,
        out_shape=(
            jax.ShapeDtypeStruct((B * V, D), jnp.float32),
            jax.ShapeDtypeStruct((1, T), jnp.int32),
        ),
        in_specs=[
            pl.BlockSpec(memory_space=pltpu.SMEM),
            pl.BlockSpec(memory_space=pltpu.VMEM),
            pl.BlockSpec(memory_space=pltpu.VMEM),
        ],
        out_specs=(
            pl.BlockSpec(memory_space=pltpu.VMEM),
            pl.BlockSpec(memory_space=pltpu.VMEM),
        ),
    )(cu_seq_lens_q, input_ids, word_embeddings)

    gather = pl.kernel(
        functools.partial(_gather_body, nchunk, tok_per_w),
        out_type=jax.ShapeDtypeStruct((T, D), jnp.float32),
        mesh=plsc.VectorSubcoreMesh(core_axis_name="c", subcore_axis_name="s"),
        scratch_types=(
            [pltpu.VMEM((tok_per_w,), jnp.int32)]
            + [pltpu.VMEM((_CHUNK, D), jnp.float32) for _ in range(_NBUF)]
            + [pltpu.SemaphoreType.DMA for _ in range(2 * _NBUF)]
        ),
    )
    out = gather(comb, scaled)
    return out.reshape(1, T, D)
